# Initial kernel scaffold; baseline (speedup 1.0000x reference)
import jax
import jax.numpy as jnp
from jax.experimental import pallas as pl


def kernel(node_attr, edge_index, edge_attr, edge_sh, W1, b1, W2, b2, bn_gamma, bn_beta):
    raise NotImplementedError("write your pallas kernel here")


# speedup vs baseline: 3.0822x; 3.0822x over previous
import functools

import jax
import jax.numpy as jnp
import numpy as np
from jax import lax
from jax.experimental import pallas as pl
from jax.experimental.pallas import tpu as pltpu
from jax.experimental.pallas import tpu_sc as plsc

N_NODES = 10000
N_EDGES = 160000
D_IN = 8
D_SH = 4
D_OUT = 16
D_PAD = 16
P = D_IN * D_SH
WNUM = P * D_OUT

NC = 2
NS = 16
NW = NC * NS
BPW = N_EDGES // NW
SC_CHUNK = 1000

TILE_E = 2000


def _build_consts():
    p1r = np.zeros((D_PAD, WNUM), np.float32)
    p2r = np.zeros((D_SH, WNUM), np.float32)
    fold = np.zeros((WNUM, D_OUT), np.float32)
    inv = 1.0 / np.sqrt(P)
    for i in range(D_IN):
        for s in range(D_SH):
            p_idx = i * D_SH + s
            for o in range(D_OUT):
                c = p_idx * D_OUT + o
                p1r[i, c] = 1.0
                p2r[s, c] = 1.0
                fold[c, o] = inv
    return jnp.asarray(p1r), jnp.asarray(p2r), jnp.asarray(fold)


def _sc_gather(table, dst_idx):
    mesh = plsc.VectorSubcoreMesh(core_axis_name="c", subcore_axis_name="s")

    @functools.partial(
        pl.kernel,
        mesh=mesh,
        out_type=jax.ShapeDtypeStruct((N_EDGES, D_PAD), jnp.float32),
        compiler_params=pltpu.CompilerParams(use_tc_tiling_on_sc=False),
        scratch_types=[
            pltpu.VMEM((BPW,), jnp.int32),
            pltpu.VMEM((BPW, D_PAD), jnp.float32),
            pltpu.SemaphoreType.DMA,
        ],
    )
    def gather_k(table_hbm, idx_hbm, out_hbm, idx_v, rows_v, sem):
        wid = lax.axis_index("s") * NC + lax.axis_index("c")
        base = wid * BPW
        pltpu.sync_copy(idx_hbm.at[pl.ds(base, BPW)], idx_v)
        pltpu.async_copy(table_hbm.at[idx_v], rows_v, sem).wait()
        pltpu.sync_copy(rows_v, out_hbm.at[pl.ds(base, BPW)])

    return gather_k(table, dst_idx)


def _sc_scatter(tp, src_idx, zeros_init, ones_rows):
    mesh = plsc.VectorSubcoreMesh(core_axis_name="c", subcore_axis_name="s")

    @functools.partial(
        pl.kernel,
        mesh=mesh,
        out_type=(
            jax.ShapeDtypeStruct((NC, N_NODES, D_OUT), jnp.float32),
            jax.ShapeDtypeStruct((NC, N_NODES, D_OUT), jnp.float32),
        ),
        compiler_params=pltpu.CompilerParams(use_tc_tiling_on_sc=False),
        scratch_types=[
            pltpu.VMEM((SC_CHUNK,), jnp.int32),
            pltpu.VMEM((SC_CHUNK, D_OUT), jnp.float32),
            pltpu.VMEM((SC_CHUNK, D_OUT), jnp.float32),
            pltpu.VMEM_SHARED((N_NODES, D_OUT), jnp.float32),
            pltpu.VMEM_SHARED((N_NODES, D_OUT), jnp.float32),
        ],
    )
    def scatter_k(tp_hbm, idx_hbm, zeros_hbm, ones_hbm,
                  outsum_hbm, outcnt_hbm,
                  idx_v, rows_v, ones_v, acc_sum, acc_cnt):
        cid = lax.axis_index("c")
        sid = lax.axis_index("s")

        @pl.when(sid == 0)
        def _():
            pltpu.sync_copy(zeros_hbm, acc_sum)
            pltpu.sync_copy(zeros_hbm, acc_cnt)

        plsc.subcore_barrier()
        pltpu.sync_copy(ones_hbm, ones_v)
        wid = sid * NC + cid
        base = wid * BPW
        for k in range(BPW // SC_CHUNK):
            off = base + k * SC_CHUNK
            pltpu.sync_copy(idx_hbm.at[pl.ds(off, SC_CHUNK)], idx_v)
            pltpu.sync_copy(tp_hbm.at[pl.ds(off, SC_CHUNK)], rows_v)
            pltpu.sync_copy(rows_v, acc_sum.at[idx_v], add=True)
            pltpu.sync_copy(ones_v, acc_cnt.at[idx_v], add=True)
        plsc.subcore_barrier()

        @pl.when(sid == 0)
        def _():
            pltpu.sync_copy(acc_sum, outsum_hbm.at[cid])
            pltpu.sync_copy(acc_cnt, outcnt_hbm.at[cid])

    return scatter_k(tp, src_idx, zeros_init, ones_rows)


def _tc_compute_body(ea_ref, xg_ref, sh_ref, w1_ref, b1_ref, w2_ref,
                     b2_ref, p1r_ref, p2r_ref, fold_ref, out_ref):
    f32 = jnp.float32
    h = jnp.maximum(
        jnp.dot(ea_ref[...], w1_ref[...], preferred_element_type=f32)
        + b1_ref[...], 0.0)
    m = jnp.dot(h, w2_ref[...], preferred_element_type=f32) + b2_ref[...]
    xsr = (jnp.dot(xg_ref[...], p1r_ref[...], preferred_element_type=f32)
           * jnp.dot(sh_ref[...], p2r_ref[...], preferred_element_type=f32))
    out_ref[...] = jnp.dot(m * xsr, fold_ref[...], preferred_element_type=f32)


def _tc_compute(ea, xg, sh, w1, b1, w2, b2, p1r, p2r, fold):
    grid = (N_EDGES // TILE_E,)
    edge_spec = lambda cols: pl.BlockSpec((TILE_E, cols), lambda i: (i, 0))
    full = lambda a: pl.BlockSpec(a.shape, lambda i: (0,) * a.ndim)
    return pl.pallas_call(
        _tc_compute_body,
        grid=grid,
        in_specs=[
            edge_spec(64), edge_spec(D_PAD), edge_spec(D_SH),
            full(w1), full(b1), full(w2), full(b2),
            full(p1r), full(p2r), full(fold),
        ],
        out_specs=edge_spec(D_OUT),
        out_shape=jax.ShapeDtypeStruct((N_EDGES, D_OUT), jnp.float32),
    )(ea, xg, sh, w1, b1, w2, b2, p1r, p2r, fold)


def _tc_finalize_body(sums_ref, cnts_ref, napad_ref, g_ref, b_ref, out_ref):
    s = sums_ref[0] + sums_ref[1]
    c = cnts_ref[0] + cnts_ref[1]
    out0 = s / jnp.maximum(c, 1.0) + napad_ref[...]
    mean = jnp.mean(out0, axis=0, keepdims=True)
    var = jnp.mean((out0 - mean) ** 2, axis=0, keepdims=True)
    out_ref[...] = (out0 - mean) * lax.rsqrt(var + 1e-5) * g_ref[...] + b_ref[...]


def _tc_finalize(sums, cnts, napad, gamma, beta):
    return pl.pallas_call(
        _tc_finalize_body,
        out_shape=jax.ShapeDtypeStruct((N_NODES, D_OUT), jnp.float32),
    )(sums, cnts, napad, gamma, beta)


def kernel(node_attr, edge_index, edge_attr, edge_sh, W1, b1, W2, b2,
           bn_gamma, bn_beta):
    p1r, p2r, fold = _build_consts()
    edge_index = edge_index.astype(jnp.int32)
    edge_dst = edge_index[1]
    edge_src = edge_index[0]
    napad = jnp.pad(node_attr, ((0, 0), (0, D_PAD - D_IN)))

    xg = _sc_gather(napad, edge_dst)
    tp = _tc_compute(edge_attr, xg, edge_sh, W1, b1.reshape(1, -1),
                     W2, b2.reshape(1, -1), p1r, p2r, fold)
    zeros_init = jnp.zeros((N_NODES, D_OUT), jnp.float32)
    ones_rows = jnp.ones((SC_CHUNK, D_OUT), jnp.float32)
    sums, cnts = _sc_scatter(tp, edge_src, zeros_init, ones_rows)
    return _tc_finalize(sums, cnts, napad, bn_gamma.reshape(1, -1),
                        bn_beta.reshape(1, -1))


# speedup vs baseline: 3.7288x; 1.2098x over previous
import functools

import jax
import jax.numpy as jnp
import numpy as np
from jax import lax
from jax.experimental import pallas as pl
from jax.experimental.pallas import tpu as pltpu
from jax.experimental.pallas import tpu_sc as plsc

N_NODES = 10000
N_EDGES = 160000
D_IN = 8
D_SH = 4
D_OUT = 16
D_PAD = 16
P = D_IN * D_SH
WNUM = P * D_OUT

NC = 2
NS = 16
NW = NC * NS
BPW = N_EDGES // NW
SC_CHUNK = 1000

TILE_E = 3200


def _build_consts():
    p1r = np.zeros((D_PAD, WNUM), np.float32)
    p2r = np.zeros((D_SH, WNUM), np.float32)
    fold = np.zeros((WNUM, D_OUT), np.float32)
    inv = 1.0 / np.sqrt(P)
    for i in range(D_IN):
        for s in range(D_SH):
            p_idx = i * D_SH + s
            for o in range(D_OUT):
                c = p_idx * D_OUT + o
                p1r[i, c] = 1.0
                p2r[s, c] = 1.0
                fold[c, o] = inv
    return jnp.asarray(p1r), jnp.asarray(p2r), jnp.asarray(fold)


def _sc_gather(table, dst_idx):
    mesh = plsc.VectorSubcoreMesh(core_axis_name="c", subcore_axis_name="s")

    @functools.partial(
        pl.kernel,
        mesh=mesh,
        out_type=jax.ShapeDtypeStruct((N_EDGES, D_PAD), jnp.float32),
        compiler_params=pltpu.CompilerParams(use_tc_tiling_on_sc=False),
        scratch_types=[
            pltpu.VMEM((BPW,), jnp.int32),
            pltpu.VMEM((BPW, D_PAD), jnp.float32),
            pltpu.SemaphoreType.DMA,
        ],
    )
    def gather_k(table_hbm, idx_hbm, out_hbm, idx_v, rows_v, sem):
        wid = lax.axis_index("s") * NC + lax.axis_index("c")
        base = wid * BPW
        pltpu.sync_copy(idx_hbm.at[pl.ds(base, BPW)], idx_v)
        pltpu.async_copy(table_hbm.at[idx_v], rows_v, sem).wait()
        pltpu.sync_copy(rows_v, out_hbm.at[pl.ds(base, BPW)])

    return gather_k(table, dst_idx)


def _sc_scatter(tp, src_idx, zeros_init, ones_rows):
    mesh = plsc.VectorSubcoreMesh(core_axis_name="c", subcore_axis_name="s")

    @functools.partial(
        pl.kernel,
        mesh=mesh,
        out_type=(
            jax.ShapeDtypeStruct((NC, N_NODES, D_OUT), jnp.float32),
            jax.ShapeDtypeStruct((NC, N_NODES, D_OUT), jnp.float32),
        ),
        compiler_params=pltpu.CompilerParams(use_tc_tiling_on_sc=False),
        scratch_types=[
            pltpu.VMEM((SC_CHUNK,), jnp.int32),
            pltpu.VMEM((SC_CHUNK, D_OUT), jnp.float32),
            pltpu.VMEM((SC_CHUNK, D_OUT), jnp.float32),
            pltpu.VMEM_SHARED((N_NODES, D_OUT), jnp.float32),
            pltpu.VMEM_SHARED((N_NODES, D_OUT), jnp.float32),
        ],
    )
    def scatter_k(tp_hbm, idx_hbm, zeros_hbm, ones_hbm,
                  outsum_hbm, outcnt_hbm,
                  idx_v, rows_v, ones_v, acc_sum, acc_cnt):
        cid = lax.axis_index("c")
        sid = lax.axis_index("s")

        @pl.when(sid == 0)
        def _():
            pltpu.sync_copy(zeros_hbm, acc_sum)
            pltpu.sync_copy(zeros_hbm, acc_cnt)

        plsc.subcore_barrier()
        pltpu.sync_copy(ones_hbm, ones_v)
        wid = sid * NC + cid
        base = wid * BPW
        for k in range(BPW // SC_CHUNK):
            off = base + k * SC_CHUNK
            pltpu.sync_copy(idx_hbm.at[pl.ds(off, SC_CHUNK)], idx_v)
            pltpu.sync_copy(tp_hbm.at[pl.ds(off, SC_CHUNK)], rows_v)
            pltpu.sync_copy(rows_v, acc_sum.at[idx_v], add=True)
            pltpu.sync_copy(ones_v, acc_cnt.at[idx_v], add=True)
        plsc.subcore_barrier()

        @pl.when(sid == 0)
        def _():
            pltpu.sync_copy(acc_sum, outsum_hbm.at[cid])
            pltpu.sync_copy(acc_cnt, outcnt_hbm.at[cid])

    return scatter_k(tp, src_idx, zeros_init, ones_rows)


def _dot0(lhs_t, rhs):
    return lax.dot_general(lhs_t, rhs, (((0,), (0,)), ((), ())),
                           preferred_element_type=jnp.float32)


def _tc_compute_body(eat_ref, xg_ref, sht_ref, w1_ref, b1_ref, w2_ref,
                     b2_ref, p1r_ref, p2r_ref, fold_ref, out_ref):
    f32 = jnp.float32
    h = jnp.maximum(_dot0(eat_ref[...], w1_ref[...]) + b1_ref[...], 0.0)
    m = jnp.dot(h, w2_ref[...], preferred_element_type=f32) + b2_ref[...]
    xsr = (jnp.dot(xg_ref[...], p1r_ref[...], preferred_element_type=f32)
           * _dot0(sht_ref[...], p2r_ref[...]))
    out_ref[...] = jnp.dot(m * xsr, fold_ref[...], preferred_element_type=f32)


def _tc_compute(ea_t, xg, sh_t, w1, b1, w2, b2, p1r, p2r, fold):
    grid = (N_EDGES // TILE_E,)
    full = lambda a: pl.BlockSpec(a.shape, lambda i: (0,) * a.ndim)
    return pl.pallas_call(
        _tc_compute_body,
        grid=grid,
        in_specs=[
            pl.BlockSpec((64, TILE_E), lambda i: (0, i)),
            pl.BlockSpec((TILE_E, D_PAD), lambda i: (i, 0)),
            pl.BlockSpec((D_SH, TILE_E), lambda i: (0, i)),
            full(w1), full(b1), full(w2), full(b2),
            full(p1r), full(p2r), full(fold),
        ],
        out_specs=pl.BlockSpec((TILE_E, D_OUT), lambda i: (i, 0)),
        out_shape=jax.ShapeDtypeStruct((N_EDGES, D_OUT), jnp.float32),
    )(ea_t, xg, sh_t, w1, b1, w2, b2, p1r, p2r, fold)


def _tc_finalize_body(sums_ref, cnts_ref, napad_ref, g_ref, b_ref, out_ref):
    s = sums_ref[0] + sums_ref[1]
    c = cnts_ref[0] + cnts_ref[1]
    out0 = s / jnp.maximum(c, 1.0) + napad_ref[...]
    mean = jnp.mean(out0, axis=0, keepdims=True)
    var = jnp.mean((out0 - mean) ** 2, axis=0, keepdims=True)
    out_ref[...] = (out0 - mean) * lax.rsqrt(var + 1e-5) * g_ref[...] + b_ref[...]


def _tc_finalize(sums, cnts, napad, gamma, beta):
    return pl.pallas_call(
        _tc_finalize_body,
        out_shape=jax.ShapeDtypeStruct((N_NODES, D_OUT), jnp.float32),
    )(sums, cnts, napad, gamma, beta)


def kernel(node_attr, edge_index, edge_attr, edge_sh, W1, b1, W2, b2,
           bn_gamma, bn_beta):
    p1r, p2r, fold = _build_consts()
    edge_index = edge_index.astype(jnp.int32)
    edge_dst = edge_index[1]
    edge_src = edge_index[0]
    napad = jnp.pad(node_attr, ((0, 0), (0, D_PAD - D_IN)))

    xg = _sc_gather(napad, edge_dst)
    tp = _tc_compute(edge_attr.T, xg, edge_sh.T, W1, b1.reshape(1, -1),
                     W2, b2.reshape(1, -1), p1r, p2r, fold)
    zeros_init = jnp.zeros((N_NODES, D_OUT), jnp.float32)
    ones_rows = jnp.ones((SC_CHUNK, D_OUT), jnp.float32)
    sums, cnts = _sc_scatter(tp, edge_src, zeros_init, ones_rows)
    return _tc_finalize(sums, cnts, napad, bn_gamma.reshape(1, -1),
                        bn_beta.reshape(1, -1))
